# Initial kernel scaffold; baseline (speedup 1.0000x reference)
import functools

import jax
import jax.numpy as jnp
from jax import lax
from jax.experimental import pallas as pl
from jax.experimental.pallas import tpu as pltpu
from jax.experimental.pallas import tpu_sc as plsc

_N = 10000
_E = 320000
_D = 128
_H = 128
_A = 10
_G = 8

_NC = 2
_NS = 16
_NT = _NC * _NS
_CHUNK = 80
_EPT = _E // _NT
_CPT = _EPT // _CHUNK
_RPT = _N // _NS
_CW = 16


def _make_seg_sum(with_cnt):
  mesh = plsc.VectorSubcoreMesh(core_axis_name="c", subcore_axis_name="s")
  out_type = [jax.ShapeDtypeStruct((_NC, _N, _D), jnp.float32)]
  scratch = [
      pltpu.VMEM((_CPT, _CHUNK), jnp.int32),
      pltpu.VMEM((_CPT, _CHUNK), jnp.int32),
      pltpu.VMEM((2, _CHUNK, _D), jnp.float32),
      pltpu.VMEM_SHARED((_N, _D), jnp.float32),
      pltpu.SemaphoreType.DMA,
      pltpu.SemaphoreType.DMA,
  ]
  if with_cnt:
    out_type.append(jax.ShapeDtypeStruct((_NC, _N, _CW), jnp.float32))
    scratch += [
        pltpu.VMEM((_CHUNK, _CW), jnp.float32),
        pltpu.VMEM_SHARED((_N, _CW), jnp.float32),
    ]

  def body(*refs):
    if with_cnt:
      (h, srcr, dstr, zrow, zcnt, agg_out, cnt_out,
       src_v, dst_v, rows, agg_sh, sem0, sem1, ones_v, cnt_sh) = refs
    else:
      (h, srcr, dstr, zrow, agg_out,
       src_v, dst_v, rows, agg_sh, sem0, sem1) = refs

    cid = lax.axis_index("c")
    sid = lax.axis_index("s")
    tid = cid * _NS + sid

    pltpu.sync_copy(zrow, agg_sh.at[pl.ds(sid * _RPT, _RPT)])
    if with_cnt:
      pltpu.sync_copy(zcnt, cnt_sh.at[pl.ds(sid * _RPT, _RPT)])

      def initones(i, c):
        ones_v[i] = jnp.ones((_CW,), jnp.float32)
        return c
      lax.fori_loop(0, _CHUNK, initones, 0)

    pltpu.sync_copy(srcr.at[tid], src_v)
    pltpu.sync_copy(dstr.at[tid], dst_v)
    plsc.subcore_barrier()

    def scat(i, buf):
      pltpu.sync_copy(rows.at[buf], agg_sh.at[dst_v.at[i]], add=True)
      if with_cnt:
        pltpu.sync_copy(ones_v, cnt_sh.at[dst_v.at[i]], add=True)

    pltpu.async_copy(h.at[src_v.at[0]], rows.at[0], sem0)

    def body2(j, c):
      i0 = 2 * j
      pltpu.async_copy(h.at[src_v.at[i0 + 1]], rows.at[1], sem1)
      pltpu.make_async_copy(h.at[src_v.at[i0]], rows.at[0], sem0).wait()
      scat(i0, 0)
      pltpu.async_copy(h.at[src_v.at[i0 + 2]], rows.at[0], sem0)
      pltpu.make_async_copy(h.at[src_v.at[i0 + 1]], rows.at[1], sem1).wait()
      scat(i0 + 1, 1)
      return c
    lax.fori_loop(0, (_CPT - 1) // 2, body2, 0)

    last = _CPT - 1
    pltpu.make_async_copy(h.at[src_v.at[last]], rows.at[0], sem0).wait()
    scat(last, 0)

    plsc.subcore_barrier()

    pltpu.sync_copy(agg_sh.at[pl.ds(sid * _RPT, _RPT)],
                    agg_out.at[cid, pl.ds(sid * _RPT, _RPT)])
    if with_cnt:
      pltpu.sync_copy(cnt_sh.at[pl.ds(sid * _RPT, _RPT)],
                      cnt_out.at[cid, pl.ds(sid * _RPT, _RPT)])

  return pl.kernel(body, mesh=mesh, out_type=out_type, scratch_types=scratch)


_seg_sum_cnt = _make_seg_sum(True)
_seg_sum = _make_seg_sum(False)

_R = 1000


def _layer_body(a0, a1, c0, c1, h, wl, bl, wr, o, *, relu):
  cnt = jnp.maximum(c0[...] + c1[...], 1.0)
  mean = (a0[...] + a1[...]) / cnt
  acc = jnp.dot(mean, wl[...], preferred_element_type=jnp.float32)
  acc = acc + bl[...] + jnp.dot(h[...], wr[...],
                                preferred_element_type=jnp.float32)
  if relu:
    acc = jnp.maximum(acc, 0.0)
  o[...] = acc


def _layer_tc(a0, a1, c0, c1, h, Wl, bl2, Wr, relu):
  return pl.pallas_call(
      functools.partial(_layer_body, relu=relu),
      grid=(_N // _R,),
      in_specs=[
          pl.BlockSpec((_R, _D), lambda i: (i, 0)),
          pl.BlockSpec((_R, _D), lambda i: (i, 0)),
          pl.BlockSpec((_R, 1), lambda i: (i, 0)),
          pl.BlockSpec((_R, 1), lambda i: (i, 0)),
          pl.BlockSpec((_R, _D), lambda i: (i, 0)),
          pl.BlockSpec((_D, _H), lambda i: (0, 0)),
          pl.BlockSpec((1, _H), lambda i: (0, 0)),
          pl.BlockSpec((_D, _H), lambda i: (0, 0)),
      ],
      out_specs=pl.BlockSpec((_R, _H), lambda i: (i, 0)),
      out_shape=jax.ShapeDtypeStruct((_N, _H), jnp.float32),
  )(a0, a1, c0, c1, h, Wl, bl2, Wr)


def _pool_body(h, b, wh, bh, o, acc, accn):
  i = pl.program_id(0)

  @pl.when(i == 0)
  def _init():
    acc[...] = jnp.zeros_like(acc)
    accn[...] = jnp.zeros_like(accn)

  m = (b[...] == lax.broadcasted_iota(jnp.int32, (_R, _G), 1)
       ).astype(jnp.float32)
  acc[...] += lax.dot_general(m, h[...], (((0,), (0,)), ((), ())),
                              preferred_element_type=jnp.float32)
  accn[...] += lax.dot_general(m, jnp.ones((_R, 1), jnp.float32),
                               (((0,), (0,)), ((), ())),
                               preferred_element_type=jnp.float32)

  @pl.when(i == _N // _R - 1)
  def _fin():
    pooled = acc[...] / jnp.maximum(accn[...], 1.0)
    o[...] = jnp.dot(pooled, wh[...],
                     preferred_element_type=jnp.float32) + bh[...]


def _pool(h, b2, Wh, bh2):
  return pl.pallas_call(
      _pool_body,
      grid=(_N // _R,),
      in_specs=[
          pl.BlockSpec((_R, _H), lambda i: (i, 0)),
          pl.BlockSpec((_R, 1), lambda i: (i, 0)),
          pl.BlockSpec((_H, _A), lambda i: (0, 0)),
          pl.BlockSpec((1, _A), lambda i: (0, 0)),
      ],
      out_specs=pl.BlockSpec((_G, _A), lambda i: (0, 0)),
      out_shape=jax.ShapeDtypeStruct((_G, _A), jnp.float32),
      scratch_shapes=[
          pltpu.VMEM((_G, _H), jnp.float32),
          pltpu.VMEM((_G, 1), jnp.float32),
      ],
  )(h, b2, Wh, bh2)


def kernel(x, edge_index, batch, Wl1, bl1, Wr1, Wl2, bl2, Wr2,
           Wl3, bl3, Wr3, Wl4, bl4, Wr4, Wh, bh):
  src = edge_index[0].astype(jnp.int32).reshape(_NT, _CPT, _CHUNK)
  dst = edge_index[1].astype(jnp.int32).reshape(_NT, _CPT, _CHUNK)
  zrow = jnp.zeros((_RPT, _D), jnp.float32)
  zcnt = jnp.zeros((_RPT, _CW), jnp.float32)

  agg_p, cnt_p = _seg_sum_cnt(x, src, dst, zrow, zcnt)
  c0 = cnt_p[0, :, 0:1]
  c1 = cnt_p[1, :, 0:1]

  h = _layer_tc(agg_p[0], agg_p[1], c0, c1, x,
                Wl1, bl1.reshape(1, _H), Wr1, True)
  for (Wl, bl, Wr, relu) in ((Wl2, bl2, Wr2, True), (Wl3, bl3, Wr3, True),
                             (Wl4, bl4, Wr4, False)):
    agg_p = _seg_sum(h, src, dst, zrow)
    h = _layer_tc(agg_p[0], agg_p[1], c0, c1, h,
                  Wl, bl.reshape(1, _H), Wr, relu)

  return _pool(h, batch.astype(jnp.int32).reshape(_N, 1),
               Wh, bh.reshape(1, _A))


# speedup vs baseline: 10.5417x; 10.5417x over previous
import functools

import jax
import jax.numpy as jnp
from jax import lax
from jax.experimental import pallas as pl
from jax.experimental.pallas import tpu as pltpu
from jax.experimental.pallas import tpu_sc as plsc

_N = 10000
_E = 320000
_D = 128
_HD = 64
_H = 128
_A = 10
_G = 8

_NC = 2
_NS = 16
_CHUNK = 200
_EPT = _E // _NS
_CPT = _EPT // _CHUNK
_NP = 10240
_RPT = _NP // _NS
_CW = 16


@functools.lru_cache(maxsize=None)
def _make_seg_sum(with_cnt):
  mesh = plsc.VectorSubcoreMesh(core_axis_name="c", subcore_axis_name="s")
  out_type = [jax.ShapeDtypeStruct((_NC, _NP, _HD), jnp.float32)]
  scratch = [
      pltpu.VMEM((_CPT, _CHUNK), jnp.int32),
      pltpu.VMEM((_CPT, _CHUNK), jnp.int32),
      pltpu.VMEM((2, _CHUNK, _HD), jnp.float32),
      pltpu.VMEM_SHARED((_NP, _HD), jnp.float32),
      pltpu.SemaphoreType.DMA,
      pltpu.SemaphoreType.DMA,
  ]
  if with_cnt:
    out_type.append(jax.ShapeDtypeStruct((_NP, _CW), jnp.float32))
    scratch += [
        pltpu.VMEM((_CHUNK, _CW), jnp.float32),
        pltpu.VMEM_SHARED((_NP, _CW), jnp.float32),
    ]

  def body(*refs):
    if with_cnt:
      (hh, srcA, srcB, dstr, zrow, zcnt, agg_out, cnt_out,
       src_v, dst_v, rows, agg_sh, sem0, sem1, ones_v, cnt_sh) = refs
    else:
      (hh, srcA, srcB, dstr, zrow, agg_out,
       src_v, dst_v, rows, agg_sh, sem0, sem1) = refs

    cid = lax.axis_index("c")
    sid = lax.axis_index("s")

    pltpu.sync_copy(zrow, agg_sh.at[pl.ds(sid * _RPT, _RPT)])
    if with_cnt:
      pltpu.sync_copy(zcnt, cnt_sh.at[pl.ds(sid * _RPT, _RPT)])

      def initones(i, c):
        ones_v[i] = jnp.ones((_CW,), jnp.float32)
        return c
      lax.fori_loop(0, _CHUNK, initones, 0)

    @pl.when(cid == 0)
    def _():
      pltpu.sync_copy(srcA.at[sid], src_v)

    @pl.when(cid == 1)
    def _():
      pltpu.sync_copy(srcB.at[sid], src_v)

    pltpu.sync_copy(dstr.at[sid], dst_v)
    plsc.subcore_barrier()

    def scat(i, buf):
      pltpu.sync_copy(rows.at[buf], agg_sh.at[dst_v.at[i]], add=True)
      if with_cnt:
        @pl.when(cid == 0)
        def _():
          pltpu.sync_copy(ones_v, cnt_sh.at[dst_v.at[i]], add=True)

    pltpu.async_copy(hh.at[src_v.at[0]], rows.at[0], sem0)

    def body2(j, c):
      i0 = 2 * j
      pltpu.async_copy(hh.at[src_v.at[i0 + 1]], rows.at[1], sem1)
      pltpu.make_async_copy(hh.at[src_v.at[i0]], rows.at[0], sem0).wait()
      scat(i0, 0)

      @pl.when(j < _CPT // 2 - 1)
      def _():
        pltpu.async_copy(hh.at[src_v.at[i0 + 2]], rows.at[0], sem0)

      pltpu.make_async_copy(hh.at[src_v.at[i0 + 1]], rows.at[1], sem1).wait()
      scat(i0 + 1, 1)
      return c
    lax.fori_loop(0, _CPT // 2, body2, 0)

    plsc.subcore_barrier()

    pltpu.sync_copy(agg_sh.at[pl.ds(sid * _RPT, _RPT)],
                    agg_out.at[cid, pl.ds(sid * _RPT, _RPT)])
    if with_cnt:
      @pl.when(cid == 0)
      def _():
        pltpu.sync_copy(cnt_sh.at[pl.ds(sid * _RPT, _RPT)],
                        cnt_out.at[pl.ds(sid * _RPT, _RPT)])

  return pl.kernel(body, mesh=mesh,
                   out_type=out_type if with_cnt else out_type[0],
                   scratch_types=scratch,
                   compiler_params=pltpu.CompilerParams(
                       use_tc_tiling_on_sc=False))


_R = 1000


def _layer_body(alo, ahi, c0, h, wl, bl, wr, o, *, relu):
  cnt = jnp.maximum(c0[...], 1.0)
  mean = jnp.concatenate([alo[...], ahi[...]], axis=1) / cnt
  acc = jnp.dot(mean, wl[...], preferred_element_type=jnp.float32)
  acc = acc + bl[...] + jnp.dot(h[...], wr[...],
                                preferred_element_type=jnp.float32)
  if relu:
    acc = jnp.maximum(acc, 0.0)
  o[...] = acc


def _layer_tc(alo, ahi, c0, h, Wl, bl2, Wr, relu):
  return pl.pallas_call(
      functools.partial(_layer_body, relu=relu),
      grid=(_N // _R,),
      in_specs=[
          pl.BlockSpec((_R, _HD), lambda i: (i, 0)),
          pl.BlockSpec((_R, _HD), lambda i: (i, 0)),
          pl.BlockSpec((_R, 1), lambda i: (i, 0)),
          pl.BlockSpec((_R, _D), lambda i: (i, 0)),
          pl.BlockSpec((_D, _H), lambda i: (0, 0)),
          pl.BlockSpec((1, _H), lambda i: (0, 0)),
          pl.BlockSpec((_D, _H), lambda i: (0, 0)),
      ],
      out_specs=pl.BlockSpec((_R, _H), lambda i: (i, 0)),
      out_shape=jax.ShapeDtypeStruct((_N, _H), jnp.float32),
  )(alo, ahi, c0, h, Wl, bl2, Wr)


def _pool_body(h, b, wh, bh, o, acc, accn):
  i = pl.program_id(0)

  @pl.when(i == 0)
  def _init():
    acc[...] = jnp.zeros_like(acc)
    accn[...] = jnp.zeros_like(accn)

  m = (b[...] == lax.broadcasted_iota(jnp.int32, (_R, _G), 1)
       ).astype(jnp.float32)
  acc[...] += lax.dot_general(m, h[...], (((0,), (0,)), ((), ())),
                              preferred_element_type=jnp.float32)
  accn[...] += lax.dot_general(m, jnp.ones((_R, 1), jnp.float32),
                               (((0,), (0,)), ((), ())),
                               preferred_element_type=jnp.float32)

  @pl.when(i == _N // _R - 1)
  def _fin():
    pooled = acc[...] / jnp.maximum(accn[...], 1.0)
    o[...] = jnp.dot(pooled, wh[...],
                     preferred_element_type=jnp.float32) + bh[...]


def _pool(h, b2, Wh, bh2):
  return pl.pallas_call(
      _pool_body,
      grid=(_N // _R,),
      in_specs=[
          pl.BlockSpec((_R, _H), lambda i: (i, 0)),
          pl.BlockSpec((_R, 1), lambda i: (i, 0)),
          pl.BlockSpec((_H, _A), lambda i: (0, 0)),
          pl.BlockSpec((1, _A), lambda i: (0, 0)),
      ],
      out_specs=pl.BlockSpec((_G, _A), lambda i: (0, 0)),
      out_shape=jax.ShapeDtypeStruct((_G, _A), jnp.float32),
      scratch_shapes=[
          pltpu.VMEM((_G, _H), jnp.float32),
          pltpu.VMEM((_G, 1), jnp.float32),
      ],
  )(h, b2, Wh, bh2)


def kernel(x, edge_index, batch, Wl1, bl1, Wr1, Wl2, bl2, Wr2,
           Wl3, bl3, Wr3, Wl4, bl4, Wr4, Wh, bh):
  src = edge_index[0].astype(jnp.int32)
  dst = edge_index[1].astype(jnp.int32)
  srcA = (2 * src).reshape(_NS, _CPT, _CHUNK)
  srcB = (2 * src + 1).reshape(_NS, _CPT, _CHUNK)
  dstr = dst.reshape(_NS, _CPT, _CHUNK)
  zrow = jnp.zeros((_RPT, _HD), jnp.float32)
  zcnt = jnp.zeros((_RPT, _CW), jnp.float32)

  hh = x.reshape(2 * _N, _HD)
  agg, cnt = _make_seg_sum(True)(hh, srcA, srcB, dstr, zrow, zcnt)
  c0 = cnt[:_N, 0:1]

  h = _layer_tc(agg[0, :_N], agg[1, :_N], c0, x,
                Wl1, bl1.reshape(1, _H), Wr1, True)
  for (Wl, bl, Wr, relu) in ((Wl2, bl2, Wr2, True), (Wl3, bl3, Wr3, True),
                             (Wl4, bl4, Wr4, False)):
    agg = _make_seg_sum(False)(h.reshape(2 * _N, _HD), srcA, srcB, dstr, zrow)
    h = _layer_tc(agg[0, :_N], agg[1, :_N], c0, h,
                  Wl, bl.reshape(1, _H), Wr, relu)

  return _pool(h, batch.astype(jnp.int32).reshape(_N, 1),
               Wh, bh.reshape(1, _A))


# speedup vs baseline: 10.7231x; 1.0172x over previous
import functools

import jax
import jax.numpy as jnp
from jax import lax
from jax.experimental import pallas as pl
from jax.experimental.pallas import tpu as pltpu
from jax.experimental.pallas import tpu_sc as plsc

_N = 10000
_E = 320000
_D = 128
_HD = 64
_H = 128
_A = 10
_G = 8

_NC = 2
_NS = 16
_CHUNK = 200
_EPT = _E // _NS
_CPT = _EPT // _CHUNK
_NP = 10240
_RPT = _NP // _NS
_CW = 16


@functools.lru_cache(maxsize=None)
def _make_seg_sum(with_cnt):
  mesh = plsc.VectorSubcoreMesh(core_axis_name="c", subcore_axis_name="s")
  out_type = [jax.ShapeDtypeStruct((_NC, _NP, _HD), jnp.float32)]
  scratch = [
      pltpu.VMEM((_CPT, _CHUNK), jnp.int32),
      pltpu.VMEM((_CPT, _CHUNK), jnp.int32),
      pltpu.VMEM((2, _CHUNK, _HD), jnp.float32),
      pltpu.VMEM_SHARED((_NP, _HD), jnp.float32),
      pltpu.SemaphoreType.DMA,
      pltpu.SemaphoreType.DMA,
  ]
  if with_cnt:
    out_type.append(jax.ShapeDtypeStruct((_NP, _CW), jnp.float32))
    scratch += [
        pltpu.VMEM((_CHUNK, _CW), jnp.float32),
        pltpu.VMEM_SHARED((_NP, _CW), jnp.float32),
    ]

  def body(*refs):
    if with_cnt:
      (hh, srcA, srcB, dstr, zrow, zcnt, agg_out, cnt_out,
       src_v, dst_v, rows, agg_sh, sem0, sem1, ones_v, cnt_sh) = refs
    else:
      (hh, srcA, srcB, dstr, zrow, agg_out,
       src_v, dst_v, rows, agg_sh, sem0, sem1) = refs

    cid = lax.axis_index("c")
    sid = lax.axis_index("s")

    pltpu.sync_copy(zrow, agg_sh.at[pl.ds(sid * _RPT, _RPT)])
    if with_cnt:
      pltpu.sync_copy(zcnt, cnt_sh.at[pl.ds(sid * _RPT, _RPT)])

      def initones(i, c):
        ones_v[i] = jnp.ones((_CW,), jnp.float32)
        return c
      lax.fori_loop(0, _CHUNK, initones, 0)

    @pl.when(cid == 0)
    def _():
      pltpu.sync_copy(srcA.at[sid], src_v)

    @pl.when(cid == 1)
    def _():
      pltpu.sync_copy(srcB.at[sid], src_v)

    pltpu.sync_copy(dstr.at[sid], dst_v)
    plsc.subcore_barrier()

    def scat(i, buf):
      pltpu.sync_copy(rows.at[buf], agg_sh.at[dst_v.at[i]], add=True)
      if with_cnt:
        @pl.when(cid == 0)
        def _():
          pltpu.sync_copy(ones_v, cnt_sh.at[dst_v.at[i]], add=True)

    pltpu.async_copy(hh.at[src_v.at[0]], rows.at[0], sem0)

    def body2(j, c):
      i0 = 2 * j
      pltpu.async_copy(hh.at[src_v.at[i0 + 1]], rows.at[1], sem1)
      pltpu.make_async_copy(hh.at[src_v.at[i0]], rows.at[0], sem0).wait()
      scat(i0, 0)

      @pl.when(j < _CPT // 2 - 1)
      def _():
        pltpu.async_copy(hh.at[src_v.at[i0 + 2]], rows.at[0], sem0)

      pltpu.make_async_copy(hh.at[src_v.at[i0 + 1]], rows.at[1], sem1).wait()
      scat(i0 + 1, 1)
      return c
    lax.fori_loop(0, _CPT // 2, body2, 0)

    plsc.subcore_barrier()

    pltpu.sync_copy(agg_sh.at[pl.ds(sid * _RPT, _RPT)],
                    agg_out.at[cid, pl.ds(sid * _RPT, _RPT)])
    if with_cnt:
      @pl.when(cid == 0)
      def _():
        pltpu.sync_copy(cnt_sh.at[pl.ds(sid * _RPT, _RPT)],
                        cnt_out.at[pl.ds(sid * _RPT, _RPT)])

  return pl.kernel(body, mesh=mesh,
                   out_type=out_type if with_cnt else out_type[0],
                   scratch_types=scratch,
                   compiler_params=pltpu.CompilerParams(
                       use_tc_tiling_on_sc=False))


_R = 2000


def _layer_body(alo, ahi, c0, h, wl, bl, wr, o, *, relu):
  cnt = jnp.maximum(c0[...], 1.0)
  mean = jnp.concatenate([alo[...], ahi[...]], axis=1) / cnt
  acc = jnp.dot(mean, wl[...], preferred_element_type=jnp.float32)
  acc = acc + bl[...] + jnp.dot(h[...], wr[...],
                                preferred_element_type=jnp.float32)
  if relu:
    acc = jnp.maximum(acc, 0.0)
  o[...] = acc


def _layer_tc(alo, ahi, c0, h, Wl, bl2, Wr, relu):
  return pl.pallas_call(
      functools.partial(_layer_body, relu=relu),
      grid=(_N // _R,),
      in_specs=[
          pl.BlockSpec((_R, _HD), lambda i: (i, 0)),
          pl.BlockSpec((_R, _HD), lambda i: (i, 0)),
          pl.BlockSpec((_R, 1), lambda i: (i, 0)),
          pl.BlockSpec((_R, _D), lambda i: (i, 0)),
          pl.BlockSpec((_D, _H), lambda i: (0, 0)),
          pl.BlockSpec((1, _H), lambda i: (0, 0)),
          pl.BlockSpec((_D, _H), lambda i: (0, 0)),
      ],
      out_specs=pl.BlockSpec((_R, _H), lambda i: (i, 0)),
      out_shape=jax.ShapeDtypeStruct((_N, _H), jnp.float32),
  )(alo, ahi, c0, h, Wl, bl2, Wr)


def _pool_body(h, b, wh, bh, o, acc, accn):
  i = pl.program_id(0)

  @pl.when(i == 0)
  def _init():
    acc[...] = jnp.zeros_like(acc)
    accn[...] = jnp.zeros_like(accn)

  m = (b[...] == lax.broadcasted_iota(jnp.int32, (_R, _G), 1)
       ).astype(jnp.float32)
  acc[...] += lax.dot_general(m, h[...], (((0,), (0,)), ((), ())),
                              preferred_element_type=jnp.float32)
  accn[...] += lax.dot_general(m, jnp.ones((_R, 1), jnp.float32),
                               (((0,), (0,)), ((), ())),
                               preferred_element_type=jnp.float32)

  @pl.when(i == _N // _R - 1)
  def _fin():
    pooled = acc[...] / jnp.maximum(accn[...], 1.0)
    o[...] = jnp.dot(pooled, wh[...],
                     preferred_element_type=jnp.float32) + bh[...]


def _pool(h, b2, Wh, bh2):
  return pl.pallas_call(
      _pool_body,
      grid=(_N // _R,),
      in_specs=[
          pl.BlockSpec((_R, _H), lambda i: (i, 0)),
          pl.BlockSpec((_R, 1), lambda i: (i, 0)),
          pl.BlockSpec((_H, _A), lambda i: (0, 0)),
          pl.BlockSpec((1, _A), lambda i: (0, 0)),
      ],
      out_specs=pl.BlockSpec((_G, _A), lambda i: (0, 0)),
      out_shape=jax.ShapeDtypeStruct((_G, _A), jnp.float32),
      scratch_shapes=[
          pltpu.VMEM((_G, _H), jnp.float32),
          pltpu.VMEM((_G, 1), jnp.float32),
      ],
  )(h, b2, Wh, bh2)


def kernel(x, edge_index, batch, Wl1, bl1, Wr1, Wl2, bl2, Wr2,
           Wl3, bl3, Wr3, Wl4, bl4, Wr4, Wh, bh):
  src = edge_index[0].astype(jnp.int32)
  dst = edge_index[1].astype(jnp.int32)
  srcA = (2 * src).reshape(_NS, _CPT, _CHUNK)
  srcB = (2 * src + 1).reshape(_NS, _CPT, _CHUNK)
  dstr = dst.reshape(_NS, _CPT, _CHUNK)
  zrow = jnp.zeros((_RPT, _HD), jnp.float32)
  zcnt = jnp.zeros((_RPT, _CW), jnp.float32)

  hh = x.reshape(2 * _N, _HD)
  agg, cnt = _make_seg_sum(True)(hh, srcA, srcB, dstr, zrow, zcnt)
  c0 = cnt[:_N, 0:1]

  h = _layer_tc(agg[0, :_N], agg[1, :_N], c0, x,
                Wl1, bl1.reshape(1, _H), Wr1, True)
  for (Wl, bl, Wr, relu) in ((Wl2, bl2, Wr2, True), (Wl3, bl3, Wr3, True),
                             (Wl4, bl4, Wr4, False)):
    agg = _make_seg_sum(False)(h.reshape(2 * _N, _HD), srcA, srcB, dstr, zrow)
    h = _layer_tc(agg[0, :_N], agg[1, :_N], c0, h,
                  Wl, bl.reshape(1, _H), Wr, relu)

  return _pool(h, batch.astype(jnp.int32).reshape(_N, 1),
               Wh, bh.reshape(1, _A))


# speedup vs baseline: 10.8246x; 1.0095x over previous
import functools

import jax
import jax.numpy as jnp
from jax import lax
from jax.experimental import pallas as pl
from jax.experimental.pallas import tpu as pltpu
from jax.experimental.pallas import tpu_sc as plsc

_N = 10000
_E = 320000
_D = 128
_HD = 64
_H = 128
_A = 10
_G = 8

_NC = 2
_NS = 16
_CHUNK = 200
_EPT = _E // _NS
_CPT = _EPT // _CHUNK
_NP = 10240
_RPT = _NP // _NS
_CW = 16


@functools.lru_cache(maxsize=None)
def _make_seg_sum(with_cnt):
  mesh = plsc.VectorSubcoreMesh(core_axis_name="c", subcore_axis_name="s")
  out_type = [jax.ShapeDtypeStruct((_NC, _NP, _HD), jnp.float32)]
  scratch = [
      pltpu.VMEM((_CPT, _CHUNK), jnp.int32),
      pltpu.VMEM((_CPT, _CHUNK), jnp.int32),
      pltpu.VMEM((2, _CHUNK, _HD), jnp.float32),
      pltpu.VMEM_SHARED((_NP, _HD), jnp.float32),
      pltpu.SemaphoreType.DMA,
      pltpu.SemaphoreType.DMA,
  ]
  if with_cnt:
    out_type.append(jax.ShapeDtypeStruct((_NP, _CW), jnp.float32))
    scratch += [
        pltpu.VMEM((_CHUNK, _CW), jnp.float32),
        pltpu.VMEM_SHARED((_NP, _CW), jnp.float32),
    ]

  def body(*refs):
    if with_cnt:
      (hh, srcA, srcB, dstr, zrow, zcnt, agg_out, cnt_out,
       src_v, dst_v, rows, agg_sh, sem0, sem1, ones_v, cnt_sh) = refs
    else:
      (hh, srcA, srcB, dstr, zrow, agg_out,
       src_v, dst_v, rows, agg_sh, sem0, sem1) = refs

    cid = lax.axis_index("c")
    sid = lax.axis_index("s")

    pltpu.sync_copy(zrow, agg_sh.at[pl.ds(sid * _RPT, _RPT)])
    if with_cnt:
      pltpu.sync_copy(zcnt, cnt_sh.at[pl.ds(sid * _RPT, _RPT)])

      def initones(i, c):
        ones_v[i] = jnp.ones((_CW,), jnp.float32)
        return c
      lax.fori_loop(0, _CHUNK, initones, 0)

    @pl.when(cid == 0)
    def _():
      pltpu.sync_copy(srcA.at[sid], src_v)

    @pl.when(cid == 1)
    def _():
      pltpu.sync_copy(srcB.at[sid], src_v)

    pltpu.sync_copy(dstr.at[sid], dst_v)
    plsc.subcore_barrier()

    def scat(i, buf):
      pltpu.sync_copy(rows.at[buf], agg_sh.at[dst_v.at[i]], add=True)
      if with_cnt:
        @pl.when(cid == 0)
        def _():
          pltpu.sync_copy(ones_v, cnt_sh.at[dst_v.at[i]], add=True)

    pltpu.async_copy(hh.at[src_v.at[0]], rows.at[0], sem0)

    def body2(j, c):
      i0 = 2 * j
      pltpu.async_copy(hh.at[src_v.at[i0 + 1]], rows.at[1], sem1)
      pltpu.make_async_copy(hh.at[src_v.at[i0]], rows.at[0], sem0).wait()
      scat(i0, 0)

      @pl.when(j < _CPT // 2 - 1)
      def _():
        pltpu.async_copy(hh.at[src_v.at[i0 + 2]], rows.at[0], sem0)

      pltpu.make_async_copy(hh.at[src_v.at[i0 + 1]], rows.at[1], sem1).wait()
      scat(i0 + 1, 1)
      return c
    lax.fori_loop(0, _CPT // 2, body2, 0)

    plsc.subcore_barrier()

    pltpu.sync_copy(agg_sh.at[pl.ds(sid * _RPT, _RPT)],
                    agg_out.at[cid, pl.ds(sid * _RPT, _RPT)])
    if with_cnt:
      @pl.when(cid == 0)
      def _():
        pltpu.sync_copy(cnt_sh.at[pl.ds(sid * _RPT, _RPT)],
                        cnt_out.at[pl.ds(sid * _RPT, _RPT)])

  return pl.kernel(body, mesh=mesh,
                   out_type=out_type if with_cnt else out_type[0],
                   scratch_types=scratch,
                   compiler_params=pltpu.CompilerParams(
                       use_tc_tiling_on_sc=False))


_R = 5000


def _layer_body(alo, ahi, c0, h, wl, bl, wr, o, *, relu):
  cnt = jnp.maximum(c0[...], 1.0)
  mean = jnp.concatenate([alo[...], ahi[...]], axis=1) / cnt
  acc = jnp.dot(mean, wl[...], preferred_element_type=jnp.float32)
  acc = acc + bl[...] + jnp.dot(h[...], wr[...],
                                preferred_element_type=jnp.float32)
  if relu:
    acc = jnp.maximum(acc, 0.0)
  o[...] = acc


def _layer_tc(alo, ahi, c0, h, Wl, bl2, Wr, relu):
  return pl.pallas_call(
      functools.partial(_layer_body, relu=relu),
      grid=(_N // _R,),
      in_specs=[
          pl.BlockSpec((_R, _HD), lambda i: (i, 0)),
          pl.BlockSpec((_R, _HD), lambda i: (i, 0)),
          pl.BlockSpec((_R, 1), lambda i: (i, 0)),
          pl.BlockSpec((_R, _D), lambda i: (i, 0)),
          pl.BlockSpec((_D, _H), lambda i: (0, 0)),
          pl.BlockSpec((1, _H), lambda i: (0, 0)),
          pl.BlockSpec((_D, _H), lambda i: (0, 0)),
      ],
      out_specs=pl.BlockSpec((_R, _H), lambda i: (i, 0)),
      out_shape=jax.ShapeDtypeStruct((_N, _H), jnp.float32),
  )(alo, ahi, c0, h, Wl, bl2, Wr)


def _pool_body(h, b, wh, bh, o, acc, accn):
  i = pl.program_id(0)

  @pl.when(i == 0)
  def _init():
    acc[...] = jnp.zeros_like(acc)
    accn[...] = jnp.zeros_like(accn)

  m = (b[...] == lax.broadcasted_iota(jnp.int32, (_R, _G), 1)
       ).astype(jnp.float32)
  acc[...] += lax.dot_general(m, h[...], (((0,), (0,)), ((), ())),
                              preferred_element_type=jnp.float32)
  accn[...] += lax.dot_general(m, jnp.ones((_R, 1), jnp.float32),
                               (((0,), (0,)), ((), ())),
                               preferred_element_type=jnp.float32)

  @pl.when(i == _N // _R - 1)
  def _fin():
    pooled = acc[...] / jnp.maximum(accn[...], 1.0)
    o[...] = jnp.dot(pooled, wh[...],
                     preferred_element_type=jnp.float32) + bh[...]


def _pool(h, b2, Wh, bh2):
  return pl.pallas_call(
      _pool_body,
      grid=(_N // _R,),
      in_specs=[
          pl.BlockSpec((_R, _H), lambda i: (i, 0)),
          pl.BlockSpec((_R, 1), lambda i: (i, 0)),
          pl.BlockSpec((_H, _A), lambda i: (0, 0)),
          pl.BlockSpec((1, _A), lambda i: (0, 0)),
      ],
      out_specs=pl.BlockSpec((_G, _A), lambda i: (0, 0)),
      out_shape=jax.ShapeDtypeStruct((_G, _A), jnp.float32),
      scratch_shapes=[
          pltpu.VMEM((_G, _H), jnp.float32),
          pltpu.VMEM((_G, 1), jnp.float32),
      ],
  )(h, b2, Wh, bh2)


def kernel(x, edge_index, batch, Wl1, bl1, Wr1, Wl2, bl2, Wr2,
           Wl3, bl3, Wr3, Wl4, bl4, Wr4, Wh, bh):
  src = edge_index[0].astype(jnp.int32)
  dst = edge_index[1].astype(jnp.int32)
  srcA = (2 * src).reshape(_NS, _CPT, _CHUNK)
  srcB = (2 * src + 1).reshape(_NS, _CPT, _CHUNK)
  dstr = dst.reshape(_NS, _CPT, _CHUNK)
  zrow = jnp.zeros((_RPT, _HD), jnp.float32)
  zcnt = jnp.zeros((_RPT, _CW), jnp.float32)

  hh = x.reshape(2 * _N, _HD)
  agg, cnt = _make_seg_sum(True)(hh, srcA, srcB, dstr, zrow, zcnt)
  c0 = cnt[:_N, 0:1]

  h = _layer_tc(agg[0, :_N], agg[1, :_N], c0, x,
                Wl1, bl1.reshape(1, _H), Wr1, True)
  for (Wl, bl, Wr, relu) in ((Wl2, bl2, Wr2, True), (Wl3, bl3, Wr3, True),
                             (Wl4, bl4, Wr4, False)):
    agg = _make_seg_sum(False)(h.reshape(2 * _N, _HD), srcA, srcB, dstr, zrow)
    h = _layer_tc(agg[0, :_N], agg[1, :_N], c0, h,
                  Wl, bl.reshape(1, _H), Wr, relu)

  return _pool(h, batch.astype(jnp.int32).reshape(_N, 1),
               Wh, bh.reshape(1, _A))
